# baseline (speedup 1.0000x reference)
import jax
import jax.numpy as jnp
from jax import lax
from jax.experimental import pallas as pl
from jax.experimental.pallas import tpu as pltpu

_SIM_THRESHOLD = 0.9
_GUARD = 0.88
_EPS = 1e-08
_NB = 128
_CH = 16

_BATCH_DOT = (((2,), (2,)), ((0,), (0,)))
_BATCH_MM = (((2,), (1,)), ((0,), (0,)))
_BATCH_OUTER = (((1,), (1,)), ((0,), (0,)))


def _general_chunk(slots_ref, out_ref, mask_out_ref, ci):
    S = slots_ref.shape[1]
    sl = pl.ds(ci * _CH, _CH)
    x = slots_ref[sl]
    gram = lax.dot_general(x, x, _BATCH_DOT, preferred_element_type=jnp.float32)
    sq = jnp.sum(x * x, axis=-1, keepdims=True)
    norm_row = jnp.sqrt(jnp.swapaxes(sq, 1, 2))
    thr = lax.dot_general(
        norm_row, _SIM_THRESHOLD * norm_row, _BATCH_OUTER,
        preferred_element_type=jnp.float32,
    )
    hit = gram > thr + _SIM_THRESHOLD * _EPS
    iota_col = lax.broadcasted_iota(jnp.int32, (_CH, S, S), 2)
    iota_row = lax.broadcasted_iota(jnp.int32, (_CH, S, S), 1)
    maskf = hit.astype(jnp.float32)
    c = jnp.sum(maskf, axis=-1, keepdims=True)
    multi = c > 1.0
    single = jnp.logical_not(multi)
    min_idx = jnp.min(jnp.where(hit, iota_col, S), axis=-1, keepdims=True)
    zero_hit = hit & multi & (iota_col != min_idx)
    zeroed = jnp.any(zero_hit, axis=1, keepdims=True)
    alive = (jnp.swapaxes(c, 1, 2) > 0.0) & jnp.logical_not(zeroed)
    mask_out_ref[pl.ds(ci * _CH, _CH)] = alive.astype(jnp.float32).reshape(
        _CH, S
    )
    candidate = multi & (min_idx == iota_col)
    writer = jnp.max(jnp.where(candidate, iota_row, -1), axis=1, keepdims=True)
    p_sel = ((jnp.swapaxes(writer, 1, 2) == iota_col) & multi).astype(
        jnp.float32
    )
    n_rows = maskf / (c + _EPS)
    m_diag = jnp.where((iota_row == iota_col) & single, 1.0, 0.0)
    merge_m = m_diag + lax.dot_general(
        p_sel, n_rows, _BATCH_MM, preferred_element_type=jnp.float32
    )
    out_ref[sl] = lax.dot_general(
        merge_m, x, _BATCH_MM, preferred_element_type=jnp.float32
    )


def _merge_kernel(slots_ref, out_ref, mask_out_ref):
    nb, S, D = slots_ref.shape
    x = slots_ref[...]
    xb = x.astype(jnp.bfloat16)
    gram_b = lax.dot_general(
        xb, xb, _BATCH_DOT, preferred_element_type=jnp.float32
    )
    sq = jnp.sum(x * x, axis=-1, keepdims=True)
    sq_row = jnp.swapaxes(sq, 1, 2)
    norm_row = jnp.sqrt(sq_row)
    guard = lax.dot_general(
        norm_row, _GUARD * norm_row, _BATCH_OUTER,
        preferred_element_type=jnp.float32,
    )
    iota_col2 = lax.broadcasted_iota(jnp.int32, (1, S, S), 2)
    iota_row2 = lax.broadcasted_iota(jnp.int32, (1, S, S), 1)
    off_diag2 = iota_row2 != iota_col2
    any_multi = jnp.any((gram_b > guard) & off_diag2)
    mask_out_ref[...] = (sq_row > 9.0 * _EPS).astype(jnp.float32).reshape(
        nb, S
    )

    @pl.when(jnp.logical_not(any_multi))
    def _fast():
        out_ref[...] = x

    @pl.when(any_multi)
    def _general():
        def body(ci, carry):
            _general_chunk(slots_ref, out_ref, mask_out_ref, ci)
            return carry

        lax.fori_loop(0, nb // _CH, body, 0)


def kernel(slots):
    B, S, D = slots.shape
    grid = (B // _NB,)
    final_slots, slot_mask = pl.pallas_call(
        _merge_kernel,
        grid=grid,
        in_specs=[pl.BlockSpec((_NB, S, D), lambda i: (i, 0, 0))],
        out_specs=[
            pl.BlockSpec((_NB, S, D), lambda i: (i, 0, 0)),
            pl.BlockSpec((_NB, S), lambda i: (i, 0)),
        ],
        out_shape=[
            jax.ShapeDtypeStruct((B, S, D), jnp.float32),
            jax.ShapeDtypeStruct((B, S), jnp.float32),
        ],
    )(slots)
    return final_slots, slot_mask


# speedup vs baseline: 1.0069x; 1.0069x over previous
import jax
import jax.numpy as jnp
from jax import lax
from jax.experimental import pallas as pl
from jax.experimental.pallas import tpu as pltpu

_SIM_THRESHOLD = 0.9
_GUARD = 0.88
_EPS = 1e-08
_NB = 128
_CH = 16

_BATCH_DOT = (((2,), (2,)), ((0,), (0,)))
_BATCH_MM = (((2,), (1,)), ((0,), (0,)))
_BATCH_OUTER = (((1,), (1,)), ((0,), (0,)))


def _general_chunk(slots_ref, out_ref, mask_out_ref, ci):
    S = slots_ref.shape[1]
    sl = pl.ds(ci * _CH, _CH)
    x = slots_ref[sl]
    gram = lax.dot_general(x, x, _BATCH_DOT, preferred_element_type=jnp.float32)
    sq = jnp.sum(x * x, axis=-1, keepdims=True)
    norm_row = jnp.sqrt(jnp.swapaxes(sq, 1, 2))
    thr = lax.dot_general(
        norm_row, _SIM_THRESHOLD * norm_row, _BATCH_OUTER,
        preferred_element_type=jnp.float32,
    )
    hit = gram > thr + _SIM_THRESHOLD * _EPS
    iota_col = lax.broadcasted_iota(jnp.int32, (_CH, S, S), 2)
    iota_row = lax.broadcasted_iota(jnp.int32, (_CH, S, S), 1)
    maskf = hit.astype(jnp.float32)
    c = jnp.sum(maskf, axis=-1, keepdims=True)
    multi = c > 1.0
    single = jnp.logical_not(multi)
    min_idx = jnp.min(jnp.where(hit, iota_col, S), axis=-1, keepdims=True)
    zero_hit = hit & multi & (iota_col != min_idx)
    zeroed = jnp.any(zero_hit, axis=1, keepdims=True)
    alive = (jnp.swapaxes(c, 1, 2) > 0.0) & jnp.logical_not(zeroed)
    mask_out_ref[pl.ds(ci * _CH, _CH)] = alive.astype(jnp.float32).reshape(
        _CH, S
    )
    candidate = multi & (min_idx == iota_col)
    writer = jnp.max(jnp.where(candidate, iota_row, -1), axis=1, keepdims=True)
    p_sel = ((jnp.swapaxes(writer, 1, 2) == iota_col) & multi).astype(
        jnp.float32
    )
    n_rows = maskf / (c + _EPS)
    m_diag = jnp.where((iota_row == iota_col) & single, 1.0, 0.0)
    merge_m = m_diag + lax.dot_general(
        p_sel, n_rows, _BATCH_MM, preferred_element_type=jnp.float32
    )
    out_ref[sl] = lax.dot_general(
        merge_m, x, _BATCH_MM, preferred_element_type=jnp.float32
    )


def _merge_kernel(slots_ref, out_ref, mask_out_ref):
    nb, S, D = slots_ref.shape
    x = slots_ref[...]
    xb = x.astype(jnp.bfloat16)
    gram_b = lax.dot_general(
        xb, xb, _BATCH_DOT, preferred_element_type=jnp.float32
    )
    ones_row = jnp.ones((nb, 1, D), dtype=jnp.bfloat16)
    sq_row = lax.dot_general(
        ones_row, xb * xb, _BATCH_DOT, preferred_element_type=jnp.float32
    )
    norm_row = jnp.sqrt(sq_row)
    guard = lax.dot_general(
        norm_row, _GUARD * norm_row, _BATCH_OUTER,
        preferred_element_type=jnp.float32,
    )
    iota_col2 = lax.broadcasted_iota(jnp.int32, (1, S, S), 2)
    iota_row2 = lax.broadcasted_iota(jnp.int32, (1, S, S), 1)
    off_diag2 = iota_row2 != iota_col2
    any_multi = jnp.any((gram_b > guard) & off_diag2)
    mask_out_ref[...] = (sq_row > 9.0 * _EPS).astype(jnp.float32).reshape(
        nb, S
    )

    @pl.when(jnp.logical_not(any_multi))
    def _fast():
        out_ref[...] = x

    @pl.when(any_multi)
    def _general():
        def body(ci, carry):
            _general_chunk(slots_ref, out_ref, mask_out_ref, ci)
            return carry

        lax.fori_loop(0, nb // _CH, body, 0)


def kernel(slots):
    B, S, D = slots.shape
    grid = (B // _NB,)
    final_slots, slot_mask = pl.pallas_call(
        _merge_kernel,
        grid=grid,
        in_specs=[pl.BlockSpec((_NB, S, D), lambda i: (i, 0, 0))],
        out_specs=[
            pl.BlockSpec((_NB, S, D), lambda i: (i, 0, 0)),
            pl.BlockSpec((_NB, S), lambda i: (i, 0)),
        ],
        out_shape=[
            jax.ShapeDtypeStruct((B, S, D), jnp.float32),
            jax.ShapeDtypeStruct((B, S), jnp.float32),
        ],
    )(slots)
    return final_slots, slot_mask
